# baseline (speedup 1.0000x reference)
import numpy as np

import jax
import jax.numpy as jnp
from jax import lax
from jax.experimental import pallas as pl
from jax.experimental.pallas import tpu as pltpu
from jax.experimental.pallas import tpu_sc as plsc

E = 320000
N = 10000
D = 128

NC = 2
NS = 16
NW = NC * NS
LANES = 16
CHUNK = E // NW
NPAD = 10240
NSL = NPAD // NS
NROW = N // NS
W = 112
NFULL = CHUNK // W
TAIL = CHUNK - NFULL * W
NEG = -3.0e38
EPS = 1e-13

def _lane():
  return lax.iota(jnp.int32, LANES)

_mesh = plsc.VectorSubcoreMesh(
    core_axis_name="c", subcore_axis_name="s", num_cores=NC, num_subcores=NS)


def _take(v, idx):
  return v.at[idx].get(mode="promise_in_bounds")


def _seg_scan(vals, ix, op):
  lane = _lane()
  for sh in (1, 2, 4, 8):
    src = jnp.maximum(lane - sh, 0)
    sv = _take(vals, src)
    si = _take(ix, src)
    same = (lane >= sh) & (si == ix)
    vals = jnp.where(same, op(vals, sv), vals)
  return vals


def _last_of_run(ix):
  lane = _lane()
  nxt = _take(ix, jnp.minimum(lane + 1, LANES - 1))
  return (lane == LANES - 1) | (ix != nxt)


_BK = 2560
_GRID1 = E // _BK


def _k1_body(x_ref, wg_ref, bg_ref, wm_ref, bm_ref, gate_ref, msg_ref):
  x = x_ref[...]
  gate_ref[...] = (
      jnp.dot(x, wg_ref[...], preferred_element_type=jnp.float32)
      + bg_ref[0, 0])
  msg_ref[...] = (
      jnp.dot(x, wm_ref[...], preferred_element_type=jnp.float32)
      + bm_ref[...])


def _k1(x, Wg, bg2, Wm, bm2):
  return pl.pallas_call(
      _k1_body,
      grid=(_GRID1,),
      in_specs=[
          pl.BlockSpec((_BK, D), lambda i: (i, 0)),
          pl.BlockSpec((D, 1), lambda i: (0, 0)),
          pl.BlockSpec((1, 1), lambda i: (0, 0)),
          pl.BlockSpec((D, D), lambda i: (0, 0)),
          pl.BlockSpec((1, D), lambda i: (0, 0)),
      ],
      out_specs=[
          pl.BlockSpec((_BK, 1), lambda i: (i, 0)),
          pl.BlockSpec((_BK, D), lambda i: (i, 0)),
      ],
      out_shape=[
          jax.ShapeDtypeStruct((E, 1), jnp.float32),
          jax.ShapeDtypeStruct((E, D), jnp.float32),
      ],
  )(x, Wg, bg2, Wm, bm2)


def _k2a_body(gate_hbm, idx_hbm, mpart_hbm, g_buf, i_buf, m_tab, red, out_sl,
              shared_m):
  c = lax.axis_index("c")
  s = lax.axis_index("s")
  wid = c * NS + s
  base = wid * CHUNK
  pltpu.sync_copy(gate_hbm.at[pl.ds(base, CHUNK)], g_buf)
  pltpu.sync_copy(idx_hbm.at[pl.ds(base, CHUNK)], i_buf)

  def init(i, _):
    m_tab[pl.ds(i * LANES, LANES)] = jnp.full((LANES,), NEG, jnp.float32)
    return 0
  lax.fori_loop(0, NPAD // LANES, init, 0)

  def step(i, _):
    g = g_buf[pl.ds(i * LANES, LANES)]
    ix = i_buf[pl.ds(i * LANES, LANES)]
    g = _seg_scan(g, ix, jnp.maximum)
    last = _last_of_run(ix)
    cur = plsc.load_gather(m_tab, [ix], mask=last)
    plsc.store_scatter(m_tab, [ix], jnp.maximum(cur, g), mask=last)
    return 0
  lax.fori_loop(0, CHUNK // LANES, step, 0)

  pltpu.sync_copy(m_tab, shared_m.at[s])
  plsc.subcore_barrier()
  pltpu.sync_copy(shared_m.at[:, pl.ds(s * NSL, NSL)], red)

  def red_step(j, _):
    acc = red[0, pl.ds(j * LANES, LANES)]
    for k in range(1, NS):
      acc = jnp.maximum(acc, red[k, pl.ds(j * LANES, LANES)])
    out_sl[pl.ds(j * LANES, LANES)] = acc
    return 0
  lax.fori_loop(0, NSL // LANES, red_step, 0)
  pltpu.sync_copy(out_sl, mpart_hbm.at[pl.ds(c * NPAD + s * NSL, NSL)])


def _k2a(gate, index):
  return pl.kernel(
      _k2a_body,
      out_type=jax.ShapeDtypeStruct((NC * NPAD,), jnp.float32),
      mesh=_mesh,
      compiler_params=pltpu.CompilerParams(needs_layout_passes=False),
      scratch_types=[
          pltpu.VMEM((CHUNK,), jnp.float32),
          pltpu.VMEM((CHUNK,), jnp.int32),
          pltpu.VMEM((NPAD,), jnp.float32),
          pltpu.VMEM((NS, NSL), jnp.float32),
          pltpu.VMEM((NSL,), jnp.float32),
          pltpu.VMEM_SHARED((NS, NPAD), jnp.float32),
      ],
  )(gate, index)


def _k2c_body(gate_hbm, idx_hbm, w_hbm, mpart_hbm, t_hbm, spart_hbm,
              g_buf, i_buf, w_buf, t_buf, m_tab, s_tab, red, out_sl, shared_s):
  c = lax.axis_index("c")
  s = lax.axis_index("s")
  wid = c * NS + s
  base = wid * CHUNK
  pltpu.sync_copy(gate_hbm.at[pl.ds(base, CHUNK)], g_buf)
  pltpu.sync_copy(idx_hbm.at[pl.ds(base, CHUNK)], i_buf)
  pltpu.sync_copy(w_hbm.at[pl.ds(base, CHUNK)], w_buf)
  pltpu.sync_copy(mpart_hbm.at[pl.ds(0, NPAD)], m_tab)
  pltpu.sync_copy(mpart_hbm.at[pl.ds(NPAD, NPAD)], s_tab)

  def minit(i, _):
    sl = pl.ds(i * LANES, LANES)
    m_tab[sl] = jnp.maximum(m_tab[sl], s_tab[sl])
    s_tab[sl] = jnp.zeros((LANES,), jnp.float32)
    return 0
  lax.fori_loop(0, NPAD // LANES, minit, 0)

  def step(i, _):
    sl = pl.ds(i * LANES, LANES)
    g = g_buf[sl]
    ix = i_buf[sl]
    w = w_buf[sl]
    mx = plsc.load_gather(m_tab, [ix])
    t = w * jnp.exp(g - mx)
    t_buf[sl] = t
    t = _seg_scan(t, ix, lambda a, b: a + b)
    last = _last_of_run(ix)
    cur = plsc.load_gather(s_tab, [ix], mask=last)
    plsc.store_scatter(s_tab, [ix], cur + t, mask=last)
    return 0
  lax.fori_loop(0, CHUNK // LANES, step, 0)

  pltpu.sync_copy(t_buf, t_hbm.at[pl.ds(base, CHUNK)])

  pltpu.sync_copy(s_tab, shared_s.at[s])
  plsc.subcore_barrier()
  pltpu.sync_copy(shared_s.at[:, pl.ds(s * NSL, NSL)], red)

  def red_step(j, _):
    acc = red[0, pl.ds(j * LANES, LANES)]
    for k in range(1, NS):
      acc = acc + red[k, pl.ds(j * LANES, LANES)]
    out_sl[pl.ds(j * LANES, LANES)] = acc
    return 0
  lax.fori_loop(0, NSL // LANES, red_step, 0)
  pltpu.sync_copy(out_sl, spart_hbm.at[pl.ds(c * NPAD + s * NSL, NSL)])


def _k2c(gate, index, w_flat, m_part):
  return pl.kernel(
      _k2c_body,
      out_type=(
          jax.ShapeDtypeStruct((E,), jnp.float32),
          jax.ShapeDtypeStruct((NC * NPAD,), jnp.float32),
      ),
      mesh=_mesh,
      compiler_params=pltpu.CompilerParams(needs_layout_passes=False),
      scratch_types=[
          pltpu.VMEM((CHUNK,), jnp.float32),
          pltpu.VMEM((CHUNK,), jnp.int32),
          pltpu.VMEM((CHUNK,), jnp.float32),
          pltpu.VMEM((CHUNK,), jnp.float32),
          pltpu.VMEM((NPAD,), jnp.float32),
          pltpu.VMEM((NPAD,), jnp.float32),
          pltpu.VMEM((NS, NSL), jnp.float32),
          pltpu.VMEM((NSL,), jnp.float32),
          pltpu.VMEM_SHARED((NS, NPAD), jnp.float32),
      ],
  )(gate, index, w_flat, m_part)


def _k3_body(msg_hbm, t_hbm, idx_hbm, spart_hbm, opart_hbm,
             cf_all, ix_all, ix_w2, ix_t, rows2, out_tab, sem_in, sem_sc):
  c = lax.axis_index("c")
  s = lax.axis_index("s")
  wid = c * NS + s
  base = wid * CHUNK

  pltpu.sync_copy(t_hbm.at[pl.ds(base, CHUNK)], cf_all)
  pltpu.sync_copy(idx_hbm.at[pl.ds(base, CHUNK)], ix_all)
  pltpu.sync_copy(spart_hbm.at[pl.ds(0, NPAD // D), :],
                  rows2.at[0, pl.ds(0, NPAD // D), :])
  pltpu.sync_copy(spart_hbm.at[pl.ds(NPAD // D, NPAD // D), :],
                  rows2.at[1, pl.ds(0, NPAD // D), :])

  def coef_step(j, _):
    sl = pl.ds(j * LANES, LANES)
    ix = ix_all[sl]
    r, q = ix >> 7, ix & (D - 1)
    zero = jnp.zeros((LANES,), jnp.int32)
    sv = (plsc.load_gather(rows2, [zero, r, q])
          + plsc.load_gather(rows2, [zero + 1, r, q]))
    cf_all[sl] = cf_all[sl] / (sv + EPS)
    return 0
  lax.fori_loop(0, CHUNK // LANES, coef_step, 0)

  def zrow(i, _):
    def zcol(j, _):
      rows2[0, i, pl.ds(j * LANES, LANES)] = jnp.zeros((LANES,), jnp.float32)
      return 0
    lax.fori_loop(0, D // LANES, zcol, 0)
    return 0
  lax.fori_loop(0, W, zrow, 0)
  r0 = s * 640

  def zero_rows(start, n):
    full, rem = n // W, n % W
    for z in range(full):
      pltpu.sync_copy(rows2.at[0], out_tab.at[pl.ds(start + z * W, W), :])
    if rem:
      pltpu.sync_copy(rows2.at[0, pl.ds(0, rem), :],
                      out_tab.at[pl.ds(start + full * W, rem), :])

  zero_rows(r0, 400)

  @pl.when(s < NS - 1)
  def _():
    zero_rows(r0 + 400, 240)
  plsc.subcore_barrier()

  def issue_in(wi, b):
    e0 = base + wi * W
    pltpu.async_copy(msg_hbm.at[pl.ds(e0, W), :], rows2.at[b], sem_in.at[b])

  def wait_in(wi, b):
    e0 = base + wi * W
    pltpu.make_async_copy(msg_hbm.at[pl.ds(e0, W), :], rows2.at[b],
                          sem_in.at[b]).wait()

  def issue_sc(b):
    pltpu.async_copy(rows2.at[b], out_tab.at[ix_w2.at[b]], sem_sc.at[b],
                     add=True)

  def wait_sc(b):
    pltpu.make_async_copy(rows2.at[b], out_tab.at[ix_w2.at[b]],
                          sem_sc.at[b]).wait()

  def compute(wi, b, nw):
    def ixcp(j, _):
      ix_w2[b, pl.ds(j * LANES, LANES)] = (
          ix_all[pl.ds(wi * W + j * LANES, LANES)])
      return 0
    lax.fori_loop(0, nw // LANES, ixcp, 0)

    def scale(e4, _):
      for u in range(4):
        e = e4 * 4 + u
        cf = plsc.load_gather(
            cf_all, [jnp.broadcast_to(wi * W + e, (LANES,))])
        for k in range(D // LANES):
          sl = pl.ds(k * LANES, LANES)
          rows2[b, e, sl] = rows2[b, e, sl] * cf
      return 0
    lax.fori_loop(0, nw // 4, scale, 0)

  issue_in(0, 0)
  issue_in(1, 1)

  def outer(j, _):
    for b in (0, 1):
      wi = j * 2 + b
      wait_in(wi, b)
      compute(wi, b, W)
      issue_sc(b)

      @pl.when(wi >= 1)
      def _():
        wait_sc(1 - b)

        @pl.when(wi + 1 < NFULL)
        def _():
          issue_in(wi + 1, 1 - b)
    return 0
  lax.fori_loop(0, NFULL // 2, outer, 0)

  last = NFULL - 1
  wait_in(last, 0)
  compute(last, 0, W)
  issue_sc(0)
  wait_sc(1)

  e0 = base + NFULL * W
  pltpu.sync_copy(msg_hbm.at[pl.ds(e0, TAIL), :],
                  rows2.at[1, pl.ds(0, TAIL), :])

  def tscale(e, _):
    cf = plsc.load_gather(
        cf_all, [jnp.broadcast_to(NFULL * W + e, (LANES,))])
    for k in range(D // LANES):
      sl = pl.ds(k * LANES, LANES)
      rows2[1, e, sl] = rows2[1, e, sl] * cf
    return 0
  lax.fori_loop(0, TAIL, tscale, 0)
  for j in range(TAIL // LANES):
    ix_t[pl.ds(j * LANES, LANES)] = ix_all[pl.ds(NFULL * W + j * LANES,
                                                 LANES)]
  pltpu.sync_copy(rows2.at[1, pl.ds(0, TAIL), :], out_tab.at[ix_t], add=True)
  wait_sc(0)

  plsc.subcore_barrier()
  pltpu.sync_copy(out_tab.at[pl.ds(r0, 400), :],
                  opart_hbm.at[pl.ds(c * N + r0, 400), :])

  @pl.when(s < NS - 1)
  def _():
    pltpu.sync_copy(out_tab.at[pl.ds(r0 + 400, 240), :],
                    opart_hbm.at[pl.ds(c * N + r0 + 400, 240), :])


def _k3(msg, t, index, s_part2d):
  return pl.kernel(
      _k3_body,
      out_type=jax.ShapeDtypeStruct((NC * N, D), jnp.float32),
      mesh=_mesh,
      compiler_params=pltpu.CompilerParams(needs_layout_passes=False),
      scratch_types=[
          pltpu.VMEM((CHUNK,), jnp.float32),
          pltpu.VMEM((CHUNK,), jnp.int32),
          pltpu.VMEM((2, W), jnp.int32),
          pltpu.VMEM((TAIL,), jnp.int32),
          pltpu.VMEM((2, W, D), jnp.float32),
          pltpu.VMEM_SHARED((N, D), jnp.float32),
          pltpu.SemaphoreType.DMA((2,)),
          pltpu.SemaphoreType.DMA((2,)),
      ],
  )(msg, t, index, s_part2d)


_BN = 1000


def _k4_body(a_ref, b_ref, o_ref):
  o_ref[...] = a_ref[...] + b_ref[...]


def _k4(a, b):
  return pl.pallas_call(
      _k4_body,
      grid=(N // _BN,),
      in_specs=[
          pl.BlockSpec((_BN, D), lambda i: (i, 0)),
          pl.BlockSpec((_BN, D), lambda i: (i, 0)),
      ],
      out_specs=pl.BlockSpec((_BN, D), lambda i: (i, 0)),
      out_shape=jax.ShapeDtypeStruct((N, D), jnp.float32),
  )(a, b)


def kernel(x, index, weights, Wg, bg, Wm, bm):
  gate2, msg = _k1(x, Wg, bg.reshape(1, 1), Wm, bm.reshape(1, D))
  gate = gate2.reshape(E)
  w_flat = weights.reshape(E)
  m_part = _k2a(gate, index)
  t, s_part = _k2c(gate, index, w_flat, m_part)
  opart = _k3(msg, t, index, s_part.reshape(NC * NPAD // D, D))
  return _k4(opart[:N], opart[N:])


# speedup vs baseline: 1.1759x; 1.1759x over previous
import numpy as np

import jax
import jax.numpy as jnp
from jax import lax
from jax.experimental import pallas as pl
from jax.experimental.pallas import tpu as pltpu
from jax.experimental.pallas import tpu_sc as plsc

E = 320000
N = 10000
D = 128

NC = 2
NS = 16
NW = NC * NS
LANES = 16
CHUNK = E // NW
NPAD = 10240
NSL = NPAD // NS
NROW = N // NS
W = 112
NFULL = CHUNK // W
TAIL = CHUNK - NFULL * W
NEG = -3.0e38
EPS = 1e-13

def _lane():
  return lax.iota(jnp.int32, LANES)

_mesh = plsc.VectorSubcoreMesh(
    core_axis_name="c", subcore_axis_name="s", num_cores=NC, num_subcores=NS)


def _take(v, idx):
  return v.at[idx].get(mode="promise_in_bounds")


def _seg_scan(vals, ix, op):
  lane = _lane()
  for sh in (1, 2, 4, 8):
    src = jnp.maximum(lane - sh, 0)
    sv = _take(vals, src)
    si = _take(ix, src)
    same = (lane >= sh) & (si == ix)
    vals = jnp.where(same, op(vals, sv), vals)
  return vals


def _last_of_run(ix):
  lane = _lane()
  nxt = _take(ix, jnp.minimum(lane + 1, LANES - 1))
  return (lane == LANES - 1) | (ix != nxt)


_BK = 2560
_GRID1 = E // _BK


def _k1_body(x_ref, wg_ref, bg_ref, wm_ref, bm_ref, gate_ref, msg_ref):
  x = x_ref[...]
  gate_ref[0] = (
      lax.dot_general(wg_ref[...], x, (((1,), (1,)), ((), ())),
                      preferred_element_type=jnp.float32)
      + bg_ref[0, 0])
  msg_ref[...] = (
      jnp.dot(x, wm_ref[...], preferred_element_type=jnp.float32)
      + bm_ref[...])


def _k1(x, WgT, bg2, Wm, bm2):
  return pl.pallas_call(
      _k1_body,
      grid=(_GRID1,),
      in_specs=[
          pl.BlockSpec((_BK, D), lambda i: (i, 0)),
          pl.BlockSpec((1, D), lambda i: (0, 0)),
          pl.BlockSpec((1, 1), lambda i: (0, 0)),
          pl.BlockSpec((D, D), lambda i: (0, 0)),
          pl.BlockSpec((1, D), lambda i: (0, 0)),
      ],
      out_specs=[
          pl.BlockSpec((1, 1, _BK), lambda i: (i, 0, 0)),
          pl.BlockSpec((_BK, D), lambda i: (i, 0)),
      ],
      out_shape=[
          jax.ShapeDtypeStruct((_GRID1, 1, _BK), jnp.float32),
          jax.ShapeDtypeStruct((E, D), jnp.float32),
      ],
  )(x, WgT, bg2, Wm, bm2)


def _k2a_body(gate_hbm, idx_hbm, mpart_hbm, g_buf, i_buf, m_tab, red, out_sl,
              shared_m):
  c = lax.axis_index("c")
  s = lax.axis_index("s")
  wid = c * NS + s
  base = wid * CHUNK
  pltpu.sync_copy(gate_hbm.at[pl.ds(base, CHUNK)], g_buf)
  pltpu.sync_copy(idx_hbm.at[pl.ds(base, CHUNK)], i_buf)

  def init(i, _):
    m_tab[pl.ds(i * LANES, LANES)] = jnp.full((LANES,), NEG, jnp.float32)
    return 0
  lax.fori_loop(0, NPAD // LANES, init, 0)

  def step(i, _):
    g = g_buf[pl.ds(i * LANES, LANES)]
    ix = i_buf[pl.ds(i * LANES, LANES)]
    g = _seg_scan(g, ix, jnp.maximum)
    last = _last_of_run(ix)
    cur = plsc.load_gather(m_tab, [ix], mask=last)
    plsc.store_scatter(m_tab, [ix], jnp.maximum(cur, g), mask=last)
    return 0
  lax.fori_loop(0, CHUNK // LANES, step, 0)

  pltpu.sync_copy(m_tab, shared_m.at[s])
  plsc.subcore_barrier()
  pltpu.sync_copy(shared_m.at[:, pl.ds(s * NSL, NSL)], red)

  def red_step(j, _):
    acc = red[0, pl.ds(j * LANES, LANES)]
    for k in range(1, NS):
      acc = jnp.maximum(acc, red[k, pl.ds(j * LANES, LANES)])
    out_sl[pl.ds(j * LANES, LANES)] = acc
    return 0
  lax.fori_loop(0, NSL // LANES, red_step, 0)
  pltpu.sync_copy(out_sl, mpart_hbm.at[pl.ds(c * NPAD + s * NSL, NSL)])


def _k2a(gate, index):
  return pl.kernel(
      _k2a_body,
      out_type=jax.ShapeDtypeStruct((NC * NPAD,), jnp.float32),
      mesh=_mesh,
      compiler_params=pltpu.CompilerParams(needs_layout_passes=False),
      scratch_types=[
          pltpu.VMEM((CHUNK,), jnp.float32),
          pltpu.VMEM((CHUNK,), jnp.int32),
          pltpu.VMEM((NPAD,), jnp.float32),
          pltpu.VMEM((NS, NSL), jnp.float32),
          pltpu.VMEM((NSL,), jnp.float32),
          pltpu.VMEM_SHARED((NS, NPAD), jnp.float32),
      ],
  )(gate, index)


def _k2c_body(gate_hbm, idx_hbm, w_hbm, mpart_hbm, t_hbm, spart_hbm,
              g_buf, i_buf, w_buf, t_buf, m_tab, s_tab, red, out_sl, shared_s):
  c = lax.axis_index("c")
  s = lax.axis_index("s")
  wid = c * NS + s
  base = wid * CHUNK
  pltpu.sync_copy(gate_hbm.at[pl.ds(base, CHUNK)], g_buf)
  pltpu.sync_copy(idx_hbm.at[pl.ds(base, CHUNK)], i_buf)
  pltpu.sync_copy(w_hbm.at[pl.ds(base, CHUNK)], w_buf)
  pltpu.sync_copy(mpart_hbm.at[pl.ds(0, NPAD)], m_tab)
  pltpu.sync_copy(mpart_hbm.at[pl.ds(NPAD, NPAD)], s_tab)

  def minit(i, _):
    sl = pl.ds(i * LANES, LANES)
    m_tab[sl] = jnp.maximum(m_tab[sl], s_tab[sl])
    s_tab[sl] = jnp.zeros((LANES,), jnp.float32)
    return 0
  lax.fori_loop(0, NPAD // LANES, minit, 0)

  def step(i, _):
    sl = pl.ds(i * LANES, LANES)
    g = g_buf[sl]
    ix = i_buf[sl]
    w = w_buf[sl]
    mx = plsc.load_gather(m_tab, [ix])
    t = w * jnp.exp(g - mx)
    t_buf[sl] = t
    t = _seg_scan(t, ix, lambda a, b: a + b)
    last = _last_of_run(ix)
    cur = plsc.load_gather(s_tab, [ix], mask=last)
    plsc.store_scatter(s_tab, [ix], cur + t, mask=last)
    return 0
  lax.fori_loop(0, CHUNK // LANES, step, 0)

  pltpu.sync_copy(t_buf, t_hbm.at[pl.ds(base, CHUNK)])

  pltpu.sync_copy(s_tab, shared_s.at[s])
  plsc.subcore_barrier()
  pltpu.sync_copy(shared_s.at[:, pl.ds(s * NSL, NSL)], red)

  def red_step(j, _):
    acc = red[0, pl.ds(j * LANES, LANES)]
    for k in range(1, NS):
      acc = acc + red[k, pl.ds(j * LANES, LANES)]
    out_sl[pl.ds(j * LANES, LANES)] = acc
    return 0
  lax.fori_loop(0, NSL // LANES, red_step, 0)
  pltpu.sync_copy(out_sl, spart_hbm.at[pl.ds(c * NPAD + s * NSL, NSL)])


def _k2c(gate, index, w_flat, m_part):
  return pl.kernel(
      _k2c_body,
      out_type=(
          jax.ShapeDtypeStruct((E,), jnp.float32),
          jax.ShapeDtypeStruct((NC * NPAD,), jnp.float32),
      ),
      mesh=_mesh,
      compiler_params=pltpu.CompilerParams(needs_layout_passes=False),
      scratch_types=[
          pltpu.VMEM((CHUNK,), jnp.float32),
          pltpu.VMEM((CHUNK,), jnp.int32),
          pltpu.VMEM((CHUNK,), jnp.float32),
          pltpu.VMEM((CHUNK,), jnp.float32),
          pltpu.VMEM((NPAD,), jnp.float32),
          pltpu.VMEM((NPAD,), jnp.float32),
          pltpu.VMEM((NS, NSL), jnp.float32),
          pltpu.VMEM((NSL,), jnp.float32),
          pltpu.VMEM_SHARED((NS, NPAD), jnp.float32),
      ],
  )(gate, index, w_flat, m_part)


def _k3_body(msg_hbm, t_hbm, idx_hbm, spart_hbm, opart_hbm,
             cf_all, ix_all, ix_w2, ix_t, rows2, out_tab, sem_in, sem_sc):
  c = lax.axis_index("c")
  s = lax.axis_index("s")
  wid = c * NS + s
  base = wid * CHUNK

  pltpu.sync_copy(t_hbm.at[pl.ds(base, CHUNK)], cf_all)
  pltpu.sync_copy(idx_hbm.at[pl.ds(base, CHUNK)], ix_all)
  pltpu.sync_copy(spart_hbm.at[pl.ds(0, NPAD // D), :],
                  rows2.at[0, pl.ds(0, NPAD // D), :])
  pltpu.sync_copy(spart_hbm.at[pl.ds(NPAD // D, NPAD // D), :],
                  rows2.at[1, pl.ds(0, NPAD // D), :])

  def coef_step(j, _):
    sl = pl.ds(j * LANES, LANES)
    ix = ix_all[sl]
    r, q = ix >> 7, ix & (D - 1)
    zero = jnp.zeros((LANES,), jnp.int32)
    sv = (plsc.load_gather(rows2, [zero, r, q])
          + plsc.load_gather(rows2, [zero + 1, r, q]))
    cf_all[sl] = cf_all[sl] / (sv + EPS)
    return 0
  lax.fori_loop(0, CHUNK // LANES, coef_step, 0)

  def zrow(i, _):
    def zcol(j, _):
      rows2[0, i, pl.ds(j * LANES, LANES)] = jnp.zeros((LANES,), jnp.float32)
      return 0
    lax.fori_loop(0, D // LANES, zcol, 0)
    return 0
  lax.fori_loop(0, W, zrow, 0)
  r0 = s * 640

  def zero_rows(start, n):
    full, rem = n // W, n % W
    for z in range(full):
      pltpu.sync_copy(rows2.at[0], out_tab.at[pl.ds(start + z * W, W), :])
    if rem:
      pltpu.sync_copy(rows2.at[0, pl.ds(0, rem), :],
                      out_tab.at[pl.ds(start + full * W, rem), :])

  zero_rows(r0, 400)

  @pl.when(s < NS - 1)
  def _():
    zero_rows(r0 + 400, 240)
  plsc.subcore_barrier()

  def issue_in(wi, b):
    e0 = base + wi * W
    pltpu.async_copy(msg_hbm.at[pl.ds(e0, W), :], rows2.at[b], sem_in.at[b])

  def wait_in(wi, b):
    e0 = base + wi * W
    pltpu.make_async_copy(msg_hbm.at[pl.ds(e0, W), :], rows2.at[b],
                          sem_in.at[b]).wait()

  def issue_sc(b):
    pltpu.async_copy(rows2.at[b], out_tab.at[ix_w2.at[b]], sem_sc.at[b],
                     add=True)

  def wait_sc(b):
    pltpu.make_async_copy(rows2.at[b], out_tab.at[ix_w2.at[b]],
                          sem_sc.at[b]).wait()

  def compute(wi, b, nw):
    def ixcp(j, _):
      ix_w2[b, pl.ds(j * LANES, LANES)] = (
          ix_all[pl.ds(wi * W + j * LANES, LANES)])
      return 0
    lax.fori_loop(0, nw // LANES, ixcp, 0)

    def scale(e4, _):
      for u in range(4):
        e = e4 * 4 + u
        cf = plsc.load_gather(
            cf_all, [jnp.broadcast_to(wi * W + e, (LANES,))])
        for k in range(D // LANES):
          sl = pl.ds(k * LANES, LANES)
          rows2[b, e, sl] = rows2[b, e, sl] * cf
      return 0
    lax.fori_loop(0, nw // 4, scale, 0)

  issue_in(0, 0)
  issue_in(1, 1)

  def outer(j, _):
    for b in (0, 1):
      wi = j * 2 + b
      wait_in(wi, b)
      compute(wi, b, W)
      issue_sc(b)

      @pl.when(wi >= 1)
      def _():
        wait_sc(1 - b)

        @pl.when(wi + 1 < NFULL)
        def _():
          issue_in(wi + 1, 1 - b)
    return 0
  lax.fori_loop(0, NFULL // 2, outer, 0)

  last = NFULL - 1
  wait_in(last, 0)
  compute(last, 0, W)
  issue_sc(0)
  wait_sc(1)

  e0 = base + NFULL * W
  pltpu.sync_copy(msg_hbm.at[pl.ds(e0, TAIL), :],
                  rows2.at[1, pl.ds(0, TAIL), :])

  def tscale(e, _):
    cf = plsc.load_gather(
        cf_all, [jnp.broadcast_to(NFULL * W + e, (LANES,))])
    for k in range(D // LANES):
      sl = pl.ds(k * LANES, LANES)
      rows2[1, e, sl] = rows2[1, e, sl] * cf
    return 0
  lax.fori_loop(0, TAIL, tscale, 0)
  for j in range(TAIL // LANES):
    ix_t[pl.ds(j * LANES, LANES)] = ix_all[pl.ds(NFULL * W + j * LANES,
                                                 LANES)]
  pltpu.sync_copy(rows2.at[1, pl.ds(0, TAIL), :], out_tab.at[ix_t], add=True)
  wait_sc(0)

  plsc.subcore_barrier()
  pltpu.sync_copy(out_tab.at[pl.ds(r0, 400), :],
                  opart_hbm.at[pl.ds(c * N + r0, 400), :])

  @pl.when(s < NS - 1)
  def _():
    pltpu.sync_copy(out_tab.at[pl.ds(r0 + 400, 240), :],
                    opart_hbm.at[pl.ds(c * N + r0 + 400, 240), :])


def _k3(msg, t, index, s_part2d):
  return pl.kernel(
      _k3_body,
      out_type=jax.ShapeDtypeStruct((NC * N, D), jnp.float32),
      mesh=_mesh,
      compiler_params=pltpu.CompilerParams(needs_layout_passes=False),
      scratch_types=[
          pltpu.VMEM((CHUNK,), jnp.float32),
          pltpu.VMEM((CHUNK,), jnp.int32),
          pltpu.VMEM((2, W), jnp.int32),
          pltpu.VMEM((TAIL,), jnp.int32),
          pltpu.VMEM((2, W, D), jnp.float32),
          pltpu.VMEM_SHARED((N, D), jnp.float32),
          pltpu.SemaphoreType.DMA((2,)),
          pltpu.SemaphoreType.DMA((2,)),
      ],
  )(msg, t, index, s_part2d)


_BN = 1000


def _k4_body(a_ref, b_ref, o_ref):
  o_ref[...] = a_ref[...] + b_ref[...]


def _k4(opart):
  nb = N // _BN
  return pl.pallas_call(
      _k4_body,
      grid=(nb,),
      in_specs=[
          pl.BlockSpec((_BN, D), lambda i: (i, 0)),
          pl.BlockSpec((_BN, D), lambda i: (i + nb, 0)),
      ],
      out_specs=pl.BlockSpec((_BN, D), lambda i: (i, 0)),
      out_shape=jax.ShapeDtypeStruct((N, D), jnp.float32),
  )(opart, opart)


def kernel(x, index, weights, Wg, bg, Wm, bm):
  gate2, msg = _k1(x, Wg.reshape(1, D), bg.reshape(1, 1), Wm,
                   bm.reshape(1, D))
  gate = gate2.reshape(E)
  w_flat = weights.reshape(E)
  m_part = _k2a(gate, index)
  t, s_part = _k2c(gate, index, w_flat, m_part)
  opart = _k3(msg, t, index, s_part.reshape(NC * NPAD // D, D))
  return _k4(opart)


# speedup vs baseline: 1.3786x; 1.1724x over previous
import numpy as np

import jax
import jax.numpy as jnp
from jax import lax
from jax.experimental import pallas as pl
from jax.experimental.pallas import tpu as pltpu
from jax.experimental.pallas import tpu_sc as plsc

E = 320000
N = 10000
D = 128

NC = 2
NS = 16
NW = NC * NS
LANES = 16
CHUNK = E // NW
NPAD = 10240
NSL = NPAD // NS
NROW = N // NS
W = 64
NFULL = CHUNK // W
TAIL = CHUNK - NFULL * W
NB = 3
NEG = -3.0e38
EPS = 1e-13

def _lane():
  return lax.iota(jnp.int32, LANES)

_mesh = plsc.VectorSubcoreMesh(
    core_axis_name="c", subcore_axis_name="s", num_cores=NC, num_subcores=NS)


def _take(v, idx):
  return v.at[idx].get(mode="promise_in_bounds")


def _seg_scan(vals, ix, op):
  lane = _lane()
  for sh in (1, 2, 4, 8):
    src = jnp.maximum(lane - sh, 0)
    sv = _take(vals, src)
    si = _take(ix, src)
    same = (lane >= sh) & (si == ix)
    vals = jnp.where(same, op(vals, sv), vals)
  return vals


def _last_of_run(ix):
  lane = _lane()
  nxt = _take(ix, jnp.minimum(lane + 1, LANES - 1))
  return (lane == LANES - 1) | (ix != nxt)


_BK = 2560
_GRID1 = E // _BK


def _k1_body(x_ref, wg_ref, bg_ref, wm_ref, bm_ref, gate_ref, msg_ref):
  x = x_ref[...]
  gate_ref[0] = (
      lax.dot_general(wg_ref[...], x, (((1,), (1,)), ((), ())),
                      preferred_element_type=jnp.float32)
      + bg_ref[0, 0])
  msg_ref[...] = (
      jnp.dot(x, wm_ref[...], preferred_element_type=jnp.float32)
      + bm_ref[...])


def _k1(x, WgT, bg2, Wm, bm2):
  return pl.pallas_call(
      _k1_body,
      grid=(_GRID1,),
      in_specs=[
          pl.BlockSpec((_BK, D), lambda i: (i, 0)),
          pl.BlockSpec((1, D), lambda i: (0, 0)),
          pl.BlockSpec((1, 1), lambda i: (0, 0)),
          pl.BlockSpec((D, D), lambda i: (0, 0)),
          pl.BlockSpec((1, D), lambda i: (0, 0)),
      ],
      out_specs=[
          pl.BlockSpec((1, 1, _BK), lambda i: (i, 0, 0)),
          pl.BlockSpec((_BK, D), lambda i: (i, 0)),
      ],
      out_shape=[
          jax.ShapeDtypeStruct((_GRID1, 1, _BK), jnp.float32),
          jax.ShapeDtypeStruct((E, D), jnp.float32),
      ],
  )(x, WgT, bg2, Wm, bm2)


def _k2a_body(gate_hbm, idx_hbm, mpart_hbm, g_buf, i_buf, m_tab, red, out_sl,
              shared_m):
  c = lax.axis_index("c")
  s = lax.axis_index("s")
  wid = c * NS + s
  base = wid * CHUNK
  pltpu.sync_copy(gate_hbm.at[pl.ds(base, CHUNK)], g_buf)
  pltpu.sync_copy(idx_hbm.at[pl.ds(base, CHUNK)], i_buf)

  def init(i, _):
    m_tab[pl.ds(i * LANES, LANES)] = jnp.full((LANES,), NEG, jnp.float32)
    return 0
  lax.fori_loop(0, NPAD // LANES, init, 0)

  def step(i, _):
    g = g_buf[pl.ds(i * LANES, LANES)]
    ix = i_buf[pl.ds(i * LANES, LANES)]
    g = _seg_scan(g, ix, jnp.maximum)
    last = _last_of_run(ix)
    cur = plsc.load_gather(m_tab, [ix], mask=last)
    plsc.store_scatter(m_tab, [ix], jnp.maximum(cur, g), mask=last)
    return 0
  lax.fori_loop(0, CHUNK // LANES, step, 0)

  pltpu.sync_copy(m_tab, shared_m.at[s])
  plsc.subcore_barrier()
  pltpu.sync_copy(shared_m.at[:, pl.ds(s * NSL, NSL)], red)

  def red_step(j, _):
    acc = red[0, pl.ds(j * LANES, LANES)]
    for k in range(1, NS):
      acc = jnp.maximum(acc, red[k, pl.ds(j * LANES, LANES)])
    out_sl[pl.ds(j * LANES, LANES)] = acc
    return 0
  lax.fori_loop(0, NSL // LANES, red_step, 0)
  pltpu.sync_copy(out_sl, mpart_hbm.at[pl.ds(c * NPAD + s * NSL, NSL)])


def _k2a(gate, index):
  return pl.kernel(
      _k2a_body,
      out_type=jax.ShapeDtypeStruct((NC * NPAD,), jnp.float32),
      mesh=_mesh,
      compiler_params=pltpu.CompilerParams(needs_layout_passes=False),
      scratch_types=[
          pltpu.VMEM((CHUNK,), jnp.float32),
          pltpu.VMEM((CHUNK,), jnp.int32),
          pltpu.VMEM((NPAD,), jnp.float32),
          pltpu.VMEM((NS, NSL), jnp.float32),
          pltpu.VMEM((NSL,), jnp.float32),
          pltpu.VMEM_SHARED((NS, NPAD), jnp.float32),
      ],
  )(gate, index)


def _k2c_body(gate_hbm, idx_hbm, w_hbm, mpart_hbm, t_hbm, spart_hbm,
              g_buf, i_buf, w_buf, t_buf, m_tab, s_tab, red, out_sl, shared_s):
  c = lax.axis_index("c")
  s = lax.axis_index("s")
  wid = c * NS + s
  base = wid * CHUNK
  pltpu.sync_copy(gate_hbm.at[pl.ds(base, CHUNK)], g_buf)
  pltpu.sync_copy(idx_hbm.at[pl.ds(base, CHUNK)], i_buf)
  pltpu.sync_copy(w_hbm.at[pl.ds(base, CHUNK)], w_buf)
  pltpu.sync_copy(mpart_hbm.at[pl.ds(0, NPAD)], m_tab)
  pltpu.sync_copy(mpart_hbm.at[pl.ds(NPAD, NPAD)], s_tab)

  def minit(i, _):
    sl = pl.ds(i * LANES, LANES)
    m_tab[sl] = jnp.maximum(m_tab[sl], s_tab[sl])
    s_tab[sl] = jnp.zeros((LANES,), jnp.float32)
    return 0
  lax.fori_loop(0, NPAD // LANES, minit, 0)

  def step(i, _):
    sl = pl.ds(i * LANES, LANES)
    g = g_buf[sl]
    ix = i_buf[sl]
    w = w_buf[sl]
    mx = plsc.load_gather(m_tab, [ix])
    t = w * jnp.exp(g - mx)
    t_buf[sl] = t
    t = _seg_scan(t, ix, lambda a, b: a + b)
    last = _last_of_run(ix)
    cur = plsc.load_gather(s_tab, [ix], mask=last)
    plsc.store_scatter(s_tab, [ix], cur + t, mask=last)
    return 0
  lax.fori_loop(0, CHUNK // LANES, step, 0)

  pltpu.sync_copy(t_buf, t_hbm.at[pl.ds(base, CHUNK)])

  pltpu.sync_copy(s_tab, shared_s.at[s])
  plsc.subcore_barrier()
  pltpu.sync_copy(shared_s.at[:, pl.ds(s * NSL, NSL)], red)

  def red_step(j, _):
    acc = red[0, pl.ds(j * LANES, LANES)]
    for k in range(1, NS):
      acc = acc + red[k, pl.ds(j * LANES, LANES)]
    out_sl[pl.ds(j * LANES, LANES)] = acc
    return 0
  lax.fori_loop(0, NSL // LANES, red_step, 0)
  pltpu.sync_copy(out_sl, spart_hbm.at[pl.ds(c * NPAD + s * NSL, NSL)])


def _k2c(gate, index, w_flat, m_part):
  return pl.kernel(
      _k2c_body,
      out_type=(
          jax.ShapeDtypeStruct((E,), jnp.float32),
          jax.ShapeDtypeStruct((NC * NPAD,), jnp.float32),
      ),
      mesh=_mesh,
      compiler_params=pltpu.CompilerParams(needs_layout_passes=False),
      scratch_types=[
          pltpu.VMEM((CHUNK,), jnp.float32),
          pltpu.VMEM((CHUNK,), jnp.int32),
          pltpu.VMEM((CHUNK,), jnp.float32),
          pltpu.VMEM((CHUNK,), jnp.float32),
          pltpu.VMEM((NPAD,), jnp.float32),
          pltpu.VMEM((NPAD,), jnp.float32),
          pltpu.VMEM((NS, NSL), jnp.float32),
          pltpu.VMEM((NSL,), jnp.float32),
          pltpu.VMEM_SHARED((NS, NPAD), jnp.float32),
      ],
  )(gate, index, w_flat, m_part)


def _k3_body(msg_hbm, t_hbm, idx_hbm, spart_hbm, opart_hbm,
             cf_all, ix_all, ix_w2, ix_t, rows2, out_tab, sem_in, sem_sc):
  c = lax.axis_index("c")
  s = lax.axis_index("s")
  wid = c * NS + s
  base = wid * CHUNK

  pltpu.sync_copy(t_hbm.at[pl.ds(base, CHUNK)], cf_all)
  pltpu.sync_copy(idx_hbm.at[pl.ds(base, CHUNK)], ix_all)
  pltpu.sync_copy(spart_hbm.at[pl.ds(0, W), :], rows2.at[0])
  pltpu.sync_copy(spart_hbm.at[pl.ds(W, NPAD // D - W), :],
                  rows2.at[2, pl.ds(0, NPAD // D - W), :])
  pltpu.sync_copy(spart_hbm.at[pl.ds(NPAD // D, W), :], rows2.at[1])
  pltpu.sync_copy(spart_hbm.at[pl.ds(NPAD // D + W, NPAD // D - W), :],
                  rows2.at[2, pl.ds(NPAD // D - W, NPAD // D - W), :])

  def coef_step(j, _):
    sl = pl.ds(j * LANES, LANES)
    ix = ix_all[sl]
    r, q = ix >> 7, ix & (D - 1)
    lo = r < W
    s0 = plsc.load_gather(
        rows2, [jnp.where(lo, 0, 2), jnp.where(lo, r, r - W), q])
    s1 = plsc.load_gather(
        rows2, [jnp.where(lo, 1, 2), jnp.where(lo, r, r - W + (NPAD // D - W)),
                q])
    cf_all[sl] = cf_all[sl] / (s0 + s1 + EPS)
    return 0
  lax.fori_loop(0, CHUNK // LANES, coef_step, 0)

  def zrow(i, _):
    def zcol(j, _):
      rows2[0, i, pl.ds(j * LANES, LANES)] = jnp.zeros((LANES,), jnp.float32)
      return 0
    lax.fori_loop(0, D // LANES, zcol, 0)
    return 0
  lax.fori_loop(0, W, zrow, 0)
  r0 = s * 640

  def zero_rows(start, n):
    full, rem = n // W, n % W
    for z in range(full):
      pltpu.sync_copy(rows2.at[0], out_tab.at[pl.ds(start + z * W, W), :])
    if rem:
      pltpu.sync_copy(rows2.at[0, pl.ds(0, rem), :],
                      out_tab.at[pl.ds(start + full * W, rem), :])

  zero_rows(r0, 400)

  @pl.when(s < NS - 1)
  def _():
    zero_rows(r0 + 400, 240)
  plsc.subcore_barrier()

  def issue_in(wi, b):
    e0 = base + wi * W
    pltpu.async_copy(msg_hbm.at[pl.ds(e0, W), :], rows2.at[b], sem_in.at[b])

  def wait_in(wi, b):
    e0 = base + wi * W
    pltpu.make_async_copy(msg_hbm.at[pl.ds(e0, W), :], rows2.at[b],
                          sem_in.at[b]).wait()

  def issue_sc(b):
    pltpu.async_copy(rows2.at[b], out_tab.at[ix_w2.at[b]], sem_sc.at[b],
                     add=True)

  def wait_sc(b):
    pltpu.make_async_copy(rows2.at[b], out_tab.at[ix_w2.at[b]],
                          sem_sc.at[b]).wait()

  def compute(wi, b, nw):
    def ixcp(j, _):
      ix_w2[b, pl.ds(j * LANES, LANES)] = (
          ix_all[pl.ds(wi * W + j * LANES, LANES)])
      return 0
    lax.fori_loop(0, nw // LANES, ixcp, 0)

    def scale(e4, _):
      for u in range(4):
        e = e4 * 4 + u
        cf = plsc.load_gather(
            cf_all, [jnp.broadcast_to(wi * W + e, (LANES,))])
        for k in range(D // LANES):
          sl = pl.ds(k * LANES, LANES)
          rows2[b, e, sl] = rows2[b, e, sl] * cf
      return 0
    lax.fori_loop(0, nw // 4, scale, 0)

  for b0 in range(NB):
    issue_in(b0, b0)

  def step(wi, b):
    wait_in(wi, b)
    compute(wi, b, W)
    issue_sc(b)

    @pl.when(wi >= 1)
    def _():
      wait_sc((b - 1) % NB)

      @pl.when(wi + 2 < NFULL)
      def _():
        issue_in(wi + 2, (b + 2) % NB)

  def outer(j, _):
    for b in range(NB):
      step(j * NB + b, b)
    return 0
  lax.fori_loop(0, NFULL // NB, outer, 0)

  e0 = base + NFULL * W
  pltpu.sync_copy(msg_hbm.at[pl.ds(e0, TAIL), :],
                  rows2.at[0, pl.ds(0, TAIL), :])

  def tscale(e, _):
    cf = plsc.load_gather(
        cf_all, [jnp.broadcast_to(NFULL * W + e, (LANES,))])
    for k in range(D // LANES):
      sl = pl.ds(k * LANES, LANES)
      rows2[0, e, sl] = rows2[0, e, sl] * cf
    return 0
  lax.fori_loop(0, TAIL, tscale, 0)
  for j in range(TAIL // LANES):
    ix_t[pl.ds(j * LANES, LANES)] = ix_all[pl.ds(NFULL * W + j * LANES,
                                                 LANES)]
  pltpu.sync_copy(rows2.at[0, pl.ds(0, TAIL), :], out_tab.at[ix_t], add=True)
  wait_sc((NFULL - 1) % NB)

  plsc.subcore_barrier()
  pltpu.sync_copy(out_tab.at[pl.ds(r0, 400), :],
                  opart_hbm.at[pl.ds(c * N + r0, 400), :])

  @pl.when(s < NS - 1)
  def _():
    pltpu.sync_copy(out_tab.at[pl.ds(r0 + 400, 240), :],
                    opart_hbm.at[pl.ds(c * N + r0 + 400, 240), :])


def _k3(msg, t, index, s_part2d):
  return pl.kernel(
      _k3_body,
      out_type=jax.ShapeDtypeStruct((NC * N, D), jnp.float32),
      mesh=_mesh,
      compiler_params=pltpu.CompilerParams(needs_layout_passes=False),
      scratch_types=[
          pltpu.VMEM((CHUNK,), jnp.float32),
          pltpu.VMEM((CHUNK,), jnp.int32),
          pltpu.VMEM((NB, W), jnp.int32),
          pltpu.VMEM((TAIL,), jnp.int32),
          pltpu.VMEM((NB, W, D), jnp.float32),
          pltpu.VMEM_SHARED((N, D), jnp.float32),
          pltpu.SemaphoreType.DMA((NB,)),
          pltpu.SemaphoreType.DMA((NB,)),
      ],
  )(msg, t, index, s_part2d)


_BN = 1000


def _k4_body(a_ref, b_ref, o_ref):
  o_ref[...] = a_ref[...] + b_ref[...]


def _k4(opart):
  nb = N // _BN
  return pl.pallas_call(
      _k4_body,
      grid=(nb,),
      in_specs=[
          pl.BlockSpec((_BN, D), lambda i: (i, 0)),
          pl.BlockSpec((_BN, D), lambda i: (i + nb, 0)),
      ],
      out_specs=pl.BlockSpec((_BN, D), lambda i: (i, 0)),
      out_shape=jax.ShapeDtypeStruct((N, D), jnp.float32),
  )(opart, opart)


def kernel(x, index, weights, Wg, bg, Wm, bm):
  gate2, msg = _k1(x, Wg.reshape(1, D), bg.reshape(1, 1), Wm,
                   bm.reshape(1, D))
  gate = gate2.reshape(E)
  w_flat = weights.reshape(E)
  m_part = _k2a(gate, index)
  t, s_part = _k2c(gate, index, w_flat, m_part)
  opart = _k3(msg, t, index, s_part.reshape(NC * NPAD // D, D))
  return _k4(opart)


# speedup vs baseline: 1.4902x; 1.0809x over previous
import numpy as np

import jax
import jax.numpy as jnp
from jax import lax
from jax.experimental import pallas as pl
from jax.experimental.pallas import tpu as pltpu
from jax.experimental.pallas import tpu_sc as plsc

E = 320000
N = 10000
D = 128

NC = 2
NS = 16
NW = NC * NS
LANES = 16
CHUNK = E // NW
NPAD = 10240
NSL = NPAD // NS
NROW = N // NS
W = 64
NFULL = CHUNK // W
TAIL = CHUNK - NFULL * W
NB = 3
NEG = -3.0e38
EPS = 1e-13

def _lane():
  return lax.iota(jnp.int32, LANES)

_mesh = plsc.VectorSubcoreMesh(
    core_axis_name="c", subcore_axis_name="s", num_cores=NC, num_subcores=NS)


def _take(v, idx):
  return v.at[idx].get(mode="promise_in_bounds")


def _seg_scan(vals, ix, op):
  lane = _lane()
  for sh in (1, 2, 4, 8):
    src = jnp.maximum(lane - sh, 0)
    sv = _take(vals, src)
    si = _take(ix, src)
    same = (lane >= sh) & (si == ix)
    vals = jnp.where(same, op(vals, sv), vals)
  return vals


def _last_of_run(ix):
  lane = _lane()
  nxt = _take(ix, jnp.minimum(lane + 1, LANES - 1))
  return (lane == LANES - 1) | (ix != nxt)


_BK = 4000
_GRID1 = E // _BK


def _k1_body(x_ref, wg_ref, bg_ref, wm_ref, bm_ref, gate_ref, msg_ref):
  x = x_ref[...]
  gate_ref[0] = (
      lax.dot_general(wg_ref[...], x, (((1,), (1,)), ((), ())),
                      preferred_element_type=jnp.float32)
      + bg_ref[0, 0])
  msg_ref[...] = (
      jnp.dot(x, wm_ref[...], preferred_element_type=jnp.float32)
      + bm_ref[...])


def _k1(x, WgT, bg2, Wm, bm2):
  return pl.pallas_call(
      _k1_body,
      grid=(_GRID1,),
      in_specs=[
          pl.BlockSpec((_BK, D), lambda i: (i, 0)),
          pl.BlockSpec((1, D), lambda i: (0, 0)),
          pl.BlockSpec((1, 1), lambda i: (0, 0)),
          pl.BlockSpec((D, D), lambda i: (0, 0)),
          pl.BlockSpec((1, D), lambda i: (0, 0)),
      ],
      out_specs=[
          pl.BlockSpec((1, 1, _BK), lambda i: (i, 0, 0)),
          pl.BlockSpec((_BK, D), lambda i: (i, 0)),
      ],
      out_shape=[
          jax.ShapeDtypeStruct((_GRID1, 1, _BK), jnp.float32),
          jax.ShapeDtypeStruct((E, D), jnp.float32),
      ],
  )(x, WgT, bg2, Wm, bm2)


def _k2a_body(gate_hbm, idx_hbm, mpart_hbm, g_buf, i_buf, m_tab, red, out_sl,
              shared_m):
  c = lax.axis_index("c")
  s = lax.axis_index("s")
  wid = c * NS + s
  base = wid * CHUNK
  pltpu.sync_copy(gate_hbm.at[pl.ds(base, CHUNK)], g_buf)
  pltpu.sync_copy(idx_hbm.at[pl.ds(base, CHUNK)], i_buf)

  def init(i, _):
    m_tab[pl.ds(i * LANES, LANES)] = jnp.full((LANES,), NEG, jnp.float32)
    return 0
  lax.fori_loop(0, NPAD // LANES, init, 0)

  def step(i, _):
    g = g_buf[pl.ds(i * LANES, LANES)]
    ix = i_buf[pl.ds(i * LANES, LANES)]
    g = _seg_scan(g, ix, jnp.maximum)
    last = _last_of_run(ix)
    cur = plsc.load_gather(m_tab, [ix], mask=last)
    plsc.store_scatter(m_tab, [ix], jnp.maximum(cur, g), mask=last)
    return 0
  lax.fori_loop(0, CHUNK // LANES, step, 0)

  pltpu.sync_copy(m_tab, shared_m.at[s])
  plsc.subcore_barrier()
  pltpu.sync_copy(shared_m.at[:, pl.ds(s * NSL, NSL)], red)

  def red_step(j, _):
    acc = red[0, pl.ds(j * LANES, LANES)]
    for k in range(1, NS):
      acc = jnp.maximum(acc, red[k, pl.ds(j * LANES, LANES)])
    out_sl[pl.ds(j * LANES, LANES)] = acc
    return 0
  lax.fori_loop(0, NSL // LANES, red_step, 0)
  pltpu.sync_copy(out_sl, mpart_hbm.at[pl.ds(c * NPAD + s * NSL, NSL)])


def _k2a(gate, index):
  return pl.kernel(
      _k2a_body,
      out_type=jax.ShapeDtypeStruct((NC * NPAD,), jnp.float32),
      mesh=_mesh,
      compiler_params=pltpu.CompilerParams(needs_layout_passes=False),
      scratch_types=[
          pltpu.VMEM((CHUNK,), jnp.float32),
          pltpu.VMEM((CHUNK,), jnp.int32),
          pltpu.VMEM((NPAD,), jnp.float32),
          pltpu.VMEM((NS, NSL), jnp.float32),
          pltpu.VMEM((NSL,), jnp.float32),
          pltpu.VMEM_SHARED((NS, NPAD), jnp.float32),
      ],
  )(gate, index)


def _k2c_body(gate_hbm, idx_hbm, w_hbm, mpart_hbm, t_hbm, spart_hbm,
              g_buf, i_buf, w_buf, t_buf, m_tab, s_tab, red, out_sl, shared_s):
  c = lax.axis_index("c")
  s = lax.axis_index("s")
  wid = c * NS + s
  base = wid * CHUNK
  pltpu.sync_copy(gate_hbm.at[pl.ds(base, CHUNK)], g_buf)
  pltpu.sync_copy(idx_hbm.at[pl.ds(base, CHUNK)], i_buf)
  pltpu.sync_copy(w_hbm.at[pl.ds(base, CHUNK)], w_buf)
  pltpu.sync_copy(mpart_hbm.at[pl.ds(0, NPAD)], m_tab)
  pltpu.sync_copy(mpart_hbm.at[pl.ds(NPAD, NPAD)], s_tab)

  def minit(i, _):
    sl = pl.ds(i * LANES, LANES)
    m_tab[sl] = jnp.maximum(m_tab[sl], s_tab[sl])
    s_tab[sl] = jnp.zeros((LANES,), jnp.float32)
    return 0
  lax.fori_loop(0, NPAD // LANES, minit, 0)

  def step(i, _):
    sl = pl.ds(i * LANES, LANES)
    g = g_buf[sl]
    ix = i_buf[sl]
    w = w_buf[sl]
    mx = plsc.load_gather(m_tab, [ix])
    t = w * jnp.exp(g - mx)
    t_buf[sl] = t
    t = _seg_scan(t, ix, lambda a, b: a + b)
    last = _last_of_run(ix)
    cur = plsc.load_gather(s_tab, [ix], mask=last)
    plsc.store_scatter(s_tab, [ix], cur + t, mask=last)
    return 0
  lax.fori_loop(0, CHUNK // LANES, step, 0)

  pltpu.sync_copy(t_buf, t_hbm.at[pl.ds(base, CHUNK)])

  pltpu.sync_copy(s_tab, shared_s.at[s])
  plsc.subcore_barrier()
  pltpu.sync_copy(shared_s.at[:, pl.ds(s * NSL, NSL)], red)

  def red_step(j, _):
    acc = red[0, pl.ds(j * LANES, LANES)]
    for k in range(1, NS):
      acc = acc + red[k, pl.ds(j * LANES, LANES)]
    out_sl[pl.ds(j * LANES, LANES)] = acc
    return 0
  lax.fori_loop(0, NSL // LANES, red_step, 0)
  pltpu.sync_copy(out_sl, spart_hbm.at[pl.ds(c * NPAD + s * NSL, NSL)])


def _k2c(gate, index, w_flat, m_part):
  return pl.kernel(
      _k2c_body,
      out_type=(
          jax.ShapeDtypeStruct((E,), jnp.float32),
          jax.ShapeDtypeStruct((NC * NPAD,), jnp.float32),
      ),
      mesh=_mesh,
      compiler_params=pltpu.CompilerParams(needs_layout_passes=False),
      scratch_types=[
          pltpu.VMEM((CHUNK,), jnp.float32),
          pltpu.VMEM((CHUNK,), jnp.int32),
          pltpu.VMEM((CHUNK,), jnp.float32),
          pltpu.VMEM((CHUNK,), jnp.float32),
          pltpu.VMEM((NPAD,), jnp.float32),
          pltpu.VMEM((NPAD,), jnp.float32),
          pltpu.VMEM((NS, NSL), jnp.float32),
          pltpu.VMEM((NSL,), jnp.float32),
          pltpu.VMEM_SHARED((NS, NPAD), jnp.float32),
      ],
  )(gate, index, w_flat, m_part)


def _k3_body(msg_hbm, t_hbm, idx_hbm, spart_hbm, opart_hbm,
             cf_all, ix_all, ix_w2, ix_t, rows2, out_tab, sem_in, sem_sc):
  c = lax.axis_index("c")
  s = lax.axis_index("s")
  wid = c * NS + s
  base = wid * CHUNK

  pltpu.sync_copy(t_hbm.at[pl.ds(base, CHUNK)], cf_all)
  pltpu.sync_copy(idx_hbm.at[pl.ds(base, CHUNK)], ix_all)
  pltpu.sync_copy(spart_hbm.at[pl.ds(0, W), :], rows2.at[0])
  pltpu.sync_copy(spart_hbm.at[pl.ds(W, NPAD // D - W), :],
                  rows2.at[2, pl.ds(0, NPAD // D - W), :])
  pltpu.sync_copy(spart_hbm.at[pl.ds(NPAD // D, W), :], rows2.at[1])
  pltpu.sync_copy(spart_hbm.at[pl.ds(NPAD // D + W, NPAD // D - W), :],
                  rows2.at[2, pl.ds(NPAD // D - W, NPAD // D - W), :])

  def coef_step(j, _):
    sl = pl.ds(j * LANES, LANES)
    ix = ix_all[sl]
    r, q = ix >> 7, ix & (D - 1)
    lo = r < W
    s0 = plsc.load_gather(
        rows2, [jnp.where(lo, 0, 2), jnp.where(lo, r, r - W), q])
    s1 = plsc.load_gather(
        rows2, [jnp.where(lo, 1, 2), jnp.where(lo, r, r - W + (NPAD // D - W)),
                q])
    cf_all[sl] = cf_all[sl] / (s0 + s1 + EPS)
    return 0
  lax.fori_loop(0, CHUNK // LANES, coef_step, 0)

  def zrow(i, _):
    def zcol(j, _):
      rows2[0, i, pl.ds(j * LANES, LANES)] = jnp.zeros((LANES,), jnp.float32)
      return 0
    lax.fori_loop(0, D // LANES, zcol, 0)
    return 0
  lax.fori_loop(0, W, zrow, 0)
  r0 = s * 640

  def zero_rows(start, n):
    full, rem = n // W, n % W
    for z in range(full):
      pltpu.sync_copy(rows2.at[0], out_tab.at[pl.ds(start + z * W, W), :])
    if rem:
      pltpu.sync_copy(rows2.at[0, pl.ds(0, rem), :],
                      out_tab.at[pl.ds(start + full * W, rem), :])

  zero_rows(r0, 400)

  @pl.when(s < NS - 1)
  def _():
    zero_rows(r0 + 400, 240)
  plsc.subcore_barrier()

  def issue_in(wi, b):
    e0 = base + wi * W
    pltpu.async_copy(msg_hbm.at[pl.ds(e0, W), :], rows2.at[b], sem_in.at[b])

  def wait_in(wi, b):
    e0 = base + wi * W
    pltpu.make_async_copy(msg_hbm.at[pl.ds(e0, W), :], rows2.at[b],
                          sem_in.at[b]).wait()

  def issue_sc(b):
    pltpu.async_copy(rows2.at[b], out_tab.at[ix_w2.at[b]], sem_sc.at[b],
                     add=True)

  def wait_sc(b):
    pltpu.make_async_copy(rows2.at[b], out_tab.at[ix_w2.at[b]],
                          sem_sc.at[b]).wait()

  def compute(wi, b, nw):
    def ixcp(j, _):
      ix_w2[b, pl.ds(j * LANES, LANES)] = (
          ix_all[pl.ds(wi * W + j * LANES, LANES)])
      return 0
    lax.fori_loop(0, nw // LANES, ixcp, 0)

    def scale(e4, _):
      for u in range(4):
        e = e4 * 4 + u
        cf = plsc.load_gather(
            cf_all, [jnp.broadcast_to(wi * W + e, (LANES,))])
        for k in range(D // LANES):
          sl = pl.ds(k * LANES, LANES)
          rows2[b, e, sl] = rows2[b, e, sl] * cf
      return 0
    lax.fori_loop(0, nw // 4, scale, 0)

  for b0 in range(NB):
    issue_in(b0, b0)

  def step(wi, b):
    wait_in(wi, b)
    compute(wi, b, W)
    issue_sc(b)

    @pl.when(wi >= 1)
    def _():
      wait_sc((b - 1) % NB)

      @pl.when(wi + 2 < NFULL)
      def _():
        issue_in(wi + 2, (b + 2) % NB)

  def outer(j, _):
    for b in range(NB):
      step(j * NB + b, b)
    return 0
  lax.fori_loop(0, NFULL // NB, outer, 0)

  e0 = base + NFULL * W
  pltpu.sync_copy(msg_hbm.at[pl.ds(e0, TAIL), :],
                  rows2.at[0, pl.ds(0, TAIL), :])

  def tscale(e, _):
    cf = plsc.load_gather(
        cf_all, [jnp.broadcast_to(NFULL * W + e, (LANES,))])
    for k in range(D // LANES):
      sl = pl.ds(k * LANES, LANES)
      rows2[0, e, sl] = rows2[0, e, sl] * cf
    return 0
  lax.fori_loop(0, TAIL, tscale, 0)
  for j in range(TAIL // LANES):
    ix_t[pl.ds(j * LANES, LANES)] = ix_all[pl.ds(NFULL * W + j * LANES,
                                                 LANES)]
  pltpu.sync_copy(rows2.at[0, pl.ds(0, TAIL), :], out_tab.at[ix_t], add=True)
  wait_sc((NFULL - 1) % NB)

  plsc.subcore_barrier()
  pltpu.sync_copy(out_tab.at[pl.ds(r0, 400), :],
                  opart_hbm.at[pl.ds(c * N + r0, 400), :])

  @pl.when(s < NS - 1)
  def _():
    pltpu.sync_copy(out_tab.at[pl.ds(r0 + 400, 240), :],
                    opart_hbm.at[pl.ds(c * N + r0 + 400, 240), :])


def _k3(msg, t, index, s_part2d):
  return pl.kernel(
      _k3_body,
      out_type=jax.ShapeDtypeStruct((NC * N, D), jnp.float32),
      mesh=_mesh,
      compiler_params=pltpu.CompilerParams(needs_layout_passes=False),
      scratch_types=[
          pltpu.VMEM((CHUNK,), jnp.float32),
          pltpu.VMEM((CHUNK,), jnp.int32),
          pltpu.VMEM((NB, W), jnp.int32),
          pltpu.VMEM((TAIL,), jnp.int32),
          pltpu.VMEM((NB, W, D), jnp.float32),
          pltpu.VMEM_SHARED((N, D), jnp.float32),
          pltpu.SemaphoreType.DMA((NB,)),
          pltpu.SemaphoreType.DMA((NB,)),
      ],
  )(msg, t, index, s_part2d)


_BN = 2000


def _k4_body(a_ref, b_ref, o_ref):
  o_ref[...] = a_ref[...] + b_ref[...]


def _k4(opart):
  nb = N // _BN
  return pl.pallas_call(
      _k4_body,
      grid=(nb,),
      in_specs=[
          pl.BlockSpec((_BN, D), lambda i: (i, 0)),
          pl.BlockSpec((_BN, D), lambda i: (i + nb, 0)),
      ],
      out_specs=pl.BlockSpec((_BN, D), lambda i: (i, 0)),
      out_shape=jax.ShapeDtypeStruct((N, D), jnp.float32),
  )(opart, opart)


def kernel(x, index, weights, Wg, bg, Wm, bm):
  gate2, msg = _k1(x, Wg.reshape(1, D), bg.reshape(1, 1), Wm,
                   bm.reshape(1, D))
  gate = gate2.reshape(E)
  w_flat = weights.reshape(E)
  m_part = _k2a(gate, index)
  t, s_part = _k2c(gate, index, w_flat, m_part)
  opart = _k3(msg, t, index, s_part.reshape(NC * NPAD // D, D))
  return _k4(opart)
